# baseline (speedup 1.0000x reference)
import functools

import jax
import jax.numpy as jnp
from jax import lax
from jax.experimental import pallas as pl
from jax.experimental.pallas import tpu as pltpu
from jax.experimental.pallas import tpu_sc as plsc

BATCH = 4096
HIST_LEN = 20
VOCAB = 1000

NUM_CORES = 2
NUM_SUBCORES = 16
NUM_WORKERS = NUM_CORES * NUM_SUBCORES
ROWS_PER_WORKER = BATCH // NUM_WORKERS
LANES = 16
BLOCKS = ROWS_PER_WORKER // LANES


def _sc_body(x_hbm, out_hbm, idx_v, hist0_v, hist1_v, sem0, sem1):
    c = lax.axis_index("c")
    s = lax.axis_index("s")
    wid = s * NUM_CORES + c
    row_base = wid * ROWS_PER_WORKER

    pltpu.sync_copy(
        x_hbm.at[pl.ds(row_base * HIST_LEN, ROWS_PER_WORKER * HIST_LEN)],
        idx_v,
    )

    lanes = lax.iota(jnp.int32, LANES)
    lane_off = lanes * VOCAB
    ones = jnp.ones((LANES,), jnp.float32)
    zeros = jnp.zeros((LANES,), jnp.float32)

    hists = [hist0_v, hist1_v]
    sems = [sem0, sem1]

    for hist in hists:
        for k in range(LANES * VOCAB // LANES):
            hist[pl.ds(k * LANES, LANES)] = zeros

    lane_row_off = lanes * HIST_LEN

    def block_addrs(r):
        return [
            lane_off
            + plsc.load_gather(idx_v, [lane_row_off + (r * LANES * HIST_LEN + l)])
            for l in range(HIST_LEN)
        ]

    copies = [None, None]
    for r in range(BLOCKS):
        b = r % 2
        hist = hists[b]
        if copies[b] is not None:
            copies[b].wait()
            for addr in block_addrs(r - 2):
                plsc.store_scatter(hist, [addr], zeros)
        for addr in block_addrs(r):
            plsc.addupdate_scatter(hist, [addr], ones)
        out_off = (row_base + r * LANES) * VOCAB
        copies[b] = pltpu.async_copy(
            hist, out_hbm.at[pl.ds(out_off, LANES * VOCAB)], sems[b]
        )
    copies[0].wait()
    copies[1].wait()


def _make_sc_kernel():
    mesh = plsc.VectorSubcoreMesh(core_axis_name="c", subcore_axis_name="s")
    return functools.partial(
        pl.kernel,
        mesh=mesh,
        out_type=jax.ShapeDtypeStruct((BATCH * VOCAB,), jnp.float32),
        scratch_types=[
            pltpu.VMEM((ROWS_PER_WORKER * HIST_LEN,), jnp.int32),
            pltpu.VMEM((LANES * VOCAB,), jnp.float32),
            pltpu.VMEM((LANES * VOCAB,), jnp.float32),
            pltpu.SemaphoreType.DMA,
            pltpu.SemaphoreType.DMA,
        ],
        compiler_params=pltpu.CompilerParams(needs_layout_passes=False),
    )(_sc_body)


_sc_kernel = _make_sc_kernel()


@jax.jit
def kernel(x):
    out_flat = _sc_kernel(x.reshape(-1))
    return out_flat.reshape(BATCH, VOCAB)


# speedup vs baseline: 2.0096x; 2.0096x over previous
import functools

import jax
import jax.numpy as jnp
from jax import lax
from jax.experimental import pallas as pl
from jax.experimental.pallas import tpu as pltpu
from jax.experimental.pallas import tpu_sc as plsc

BATCH = 4096
HIST_LEN = 20
VOCAB = 1000

NUM_CORES = 2
NUM_SUBCORES = 16
NUM_WORKERS = NUM_CORES * NUM_SUBCORES
COLS_PER_WORKER = BATCH // NUM_WORKERS
LANES = 16
GROUPS = COLS_PER_WORKER // LANES


def _sc_body(x_hbm, out_hbm, idx_v, hist_v, sem):
    c = lax.axis_index("c")
    s = lax.axis_index("s")
    wid = s * NUM_CORES + c
    col_base = wid * COLS_PER_WORKER

    pltpu.sync_copy(
        x_hbm.at[pl.ds(col_base * HIST_LEN, COLS_PER_WORKER * HIST_LEN)],
        idx_v,
    )

    lanes = lax.iota(jnp.int32, LANES)
    ones = jnp.ones((LANES,), jnp.float32)
    zeros = jnp.zeros((LANES,), jnp.float32)

    def clear_step(i, _):
        v8 = i * 8
        for dv in range(8):
            for ch in range(GROUPS):
                hist_v[v8 + dv, pl.ds(ch * LANES, LANES)] = zeros
        return 0

    lax.fori_loop(0, VOCAB // 8, clear_step, 0)

    lane_row_off = lanes * HIST_LEN
    for g in range(GROUPS):
        cols = lanes + g * LANES
        for l in range(HIST_LEN):
            v = plsc.load_gather(
                idx_v, [lane_row_off + (g * LANES * HIST_LEN + l)]
            )
            plsc.addupdate_scatter(hist_v, [v, cols], ones)

    pltpu.async_copy(
        hist_v, out_hbm.at[:, pl.ds(col_base, COLS_PER_WORKER)], sem
    ).wait()


def _make_sc_kernel():
    mesh = plsc.VectorSubcoreMesh(core_axis_name="c", subcore_axis_name="s")
    return functools.partial(
        pl.kernel,
        mesh=mesh,
        out_type=jax.ShapeDtypeStruct((VOCAB, BATCH), jnp.float32),
        scratch_types=[
            pltpu.VMEM((COLS_PER_WORKER * HIST_LEN,), jnp.int32),
            pltpu.VMEM((VOCAB, COLS_PER_WORKER), jnp.float32),
            pltpu.SemaphoreType.DMA,
        ],
        compiler_params=pltpu.CompilerParams(
            needs_layout_passes=False, use_tc_tiling_on_sc=True
        ),
    )(_sc_body)


_sc_kernel = _make_sc_kernel()


@jax.jit
def kernel(x):
    return _sc_kernel(x.reshape(-1)).T


# speedup vs baseline: 2.0300x; 1.0102x over previous
import functools

import jax
import jax.numpy as jnp
from jax import lax
from jax.experimental import pallas as pl
from jax.experimental.pallas import tpu as pltpu
from jax.experimental.pallas import tpu_sc as plsc

BATCH = 4096
HIST_LEN = 20
VOCAB = 1000

NUM_CORES = 2
NUM_SUBCORES = 16
NUM_WORKERS = NUM_CORES * NUM_SUBCORES
COLS_PER_WORKER = BATCH // NUM_WORKERS
LANES = 16
GROUPS = COLS_PER_WORKER // LANES
STAGE = VOCAB


def _sc_body(xt_hbm, xtail_hbm, out_hbm, hist_v, sem):
    c = lax.axis_index("c")
    s = lax.axis_index("s")
    wid = s * NUM_CORES + c
    col_base = wid * COLS_PER_WORKER

    lanes = lax.iota(jnp.int32, LANES)
    ones = jnp.ones((LANES,), jnp.float32)
    zeros = jnp.zeros((LANES,), jnp.float32)

    stage_copy = pltpu.async_copy(
        xt_hbm.at[pl.ds(0, 8), pl.ds(col_base, COLS_PER_WORKER)],
        hist_v.at[pl.ds(STAGE, 8), :],
        sem,
    )

    def clear_step(i, _):
        v8 = i * 8
        for dv in range(8):
            for ch in range(GROUPS):
                hist_v[v8 + dv, pl.ds(ch * LANES, LANES)] = zeros
        return 0

    lax.fori_loop(0, VOCAB // 8, clear_step, 0)
    stage_copy.wait()

    def scatter_pass(l_lo, l_hi, stage_off):
        for g in range(GROUPS):
            cols = lanes + g * LANES
            for l in range(l_lo, l_hi):
                row = jnp.full((LANES,), STAGE + (l - stage_off), jnp.int32)
                v_bits = plsc.load_gather(hist_v, [row, cols])
                v = plsc.bitcast(v_bits, jnp.int32)
                plsc.addupdate_scatter(hist_v, [v, cols], ones)

    scatter_pass(0, 8, 0)
    pltpu.sync_copy(
        xt_hbm.at[pl.ds(8, 8), pl.ds(col_base, COLS_PER_WORKER)],
        hist_v.at[pl.ds(STAGE, 8), :],
    )
    scatter_pass(8, 16, 8)
    pltpu.sync_copy(
        xtail_hbm.at[:, pl.ds(col_base, COLS_PER_WORKER)],
        hist_v.at[pl.ds(STAGE, 8), :],
    )
    scatter_pass(16, HIST_LEN, 12)

    pltpu.async_copy(
        hist_v.at[pl.ds(0, VOCAB), :],
        out_hbm.at[:, pl.ds(col_base, COLS_PER_WORKER)],
        sem,
    ).wait()


def _make_sc_kernel():
    mesh = plsc.VectorSubcoreMesh(core_axis_name="c", subcore_axis_name="s")
    return functools.partial(
        pl.kernel,
        mesh=mesh,
        out_type=jax.ShapeDtypeStruct((VOCAB, BATCH), jnp.float32),
        scratch_types=[
            pltpu.VMEM((VOCAB + 8, COLS_PER_WORKER), jnp.float32),
            pltpu.SemaphoreType.DMA,
        ],
        compiler_params=pltpu.CompilerParams(
            needs_layout_passes=False, use_tc_tiling_on_sc=True
        ),
    )(_sc_body)


_sc_kernel = _make_sc_kernel()


@jax.jit
def kernel(x):
    xt = jax.lax.bitcast_convert_type(x.T, jnp.float32)
    xtail = xt[HIST_LEN - 8 :]
    return _sc_kernel(xt, xtail).T
